# baseline (speedup 1.0000x reference)
import functools

import jax
import jax.numpy as jnp
from jax import lax
from jax.experimental import pallas as pl
from jax.experimental.pallas import tpu as pltpu
from jax.experimental.pallas import tpu_sc as plsc

B, T, S, D = 32, 8, 2048, 1024
H = D // 2
L = 16
SBLK = 1024
NS = S // SBLK
NUM_SC_CORES = 2
NUM_SC_SUBCORES = 16


def _rotate_sc_body(k_new_hbm, cos_hbm, sin_hbm, pos_hbm, rot_hbm,
                    pos_v, kv, crow, srow, rot, sem):
    c = lax.axis_index("c")
    s = lax.axis_index("s")
    b = s * NUM_SC_CORES + c

    pltpu.sync_copy(pos_hbm, pos_v)
    idx = pos_v.at[b]
    cp_c = pltpu.async_copy(cos_hbm.at[idx], crow, sem)
    cp_s = pltpu.async_copy(sin_hbm.at[idx], srow, sem)
    pltpu.sync_copy(k_new_hbm.at[b], kv)
    cp_c.wait()
    cp_s.wait()

    lane = lax.iota(jnp.int32, L)
    lane2 = lane * 2

    for r in range(T):
        rows = jnp.full((L,), r, jnp.int32)

        def chunk(j, carry, r=r, rows=rows):
            i0 = j * L
            c = crow[r, pl.ds(i0, L)]
            s = srow[r, pl.ds(i0, L)]
            ev = 2 * i0 + lane2
            od = ev + 1
            x_e = plsc.load_gather(kv, [rows, ev])
            x_o = plsc.load_gather(kv, [rows, od])
            plsc.store_scatter(rot, [rows, ev], x_e * c - x_o * s)
            plsc.store_scatter(rot, [rows, od], x_e * s + x_o * c)
            return carry

        lax.fori_loop(0, H // L, chunk, 0)

    pltpu.sync_copy(rot, rot_hbm.at[b])


def _rotate_sc(k_new, cos, sin, positions):
    mesh = plsc.VectorSubcoreMesh(core_axis_name="c", subcore_axis_name="s")
    fn = pl.kernel(
        _rotate_sc_body,
        out_type=jax.ShapeDtypeStruct((B, T, D), jnp.float32),
        mesh=mesh,
        scratch_types=[
            pltpu.VMEM((B, T), jnp.int32),
            pltpu.VMEM((T, D), jnp.float32),
            pltpu.VMEM((T, H), jnp.float32),
            pltpu.VMEM((T, H), jnp.float32),
            pltpu.VMEM((T, D), jnp.float32),
            pltpu.SemaphoreType.DMA,
        ],
        compiler_params=pltpu.CompilerParams(needs_layout_passes=False),
    )
    return fn(k_new, cos, sin, positions)


def _scatter_sc_body(rot_hbm, v_new_hbm, pos_hbm, big_ref, pos_v, rbuf, vbuf, sem):
    c = lax.axis_index("c")
    s = lax.axis_index("s")
    b = s * NUM_SC_CORES + c

    pltpu.sync_copy(pos_hbm, pos_v)
    idx = pos_v.at[b]
    cp_r = pltpu.async_copy(rot_hbm.at[b], rbuf, sem)
    cp_v = pltpu.async_copy(v_new_hbm.at[b], vbuf, sem)
    cp_r.wait()
    cp_v.wait()
    cp_rk = pltpu.async_copy(rbuf, big_ref.at[0, b].at[idx], sem)
    cp_rv = pltpu.async_copy(vbuf, big_ref.at[1, b].at[idx], sem)
    cp_rk.wait()
    cp_rv.wait()


def _scatter_sc(rot, v_new, positions, big_ref):
    mesh = plsc.VectorSubcoreMesh(core_axis_name="c", subcore_axis_name="s")
    fn = pl.kernel(
        _scatter_sc_body,
        out_type=(),
        mesh=mesh,
        scratch_types=[
            pltpu.VMEM((B, T), jnp.int32),
            pltpu.VMEM((T, D), jnp.float32),
            pltpu.VMEM((T, D), jnp.float32),
            pltpu.SemaphoreType.DMA,
        ],
        compiler_params=pltpu.CompilerParams(needs_layout_passes=False),
    )
    fn(rot, v_new, positions, big_ref)


def _copy_body(ck_ref, cv_ref, out_ref):
    out_ref[0, 0] = ck_ref[0]
    out_ref[1, 0] = cv_ref[0]


def _copy_tc(cache_k, cache_v):
    return pl.pallas_call(
        _copy_body,
        grid=(B, NS),
        in_specs=[
            pl.BlockSpec((1, SBLK, D), lambda b, si: (b, si, 0)),
            pl.BlockSpec((1, SBLK, D), lambda b, si: (b, si, 0)),
        ],
        out_specs=pl.BlockSpec((2, 1, SBLK, D), lambda b, si: (0, b, si, 0)),
        out_shape=jax.ShapeDtypeStruct((2, B, S, D), jnp.float32),
        compiler_params=pltpu.CompilerParams(
            dimension_semantics=("arbitrary", "arbitrary"),
        ),
    )(cache_k, cache_v)


def kernel(k_new, v_new, cos, sin, cache_k, cache_v, positions):
    rotated = _rotate_sc(k_new, cos, sin, positions)
    stacked = _copy_tc(cache_k, cache_v)
    big_ref = jax.new_ref(stacked)
    _scatter_sc(rotated, v_new, positions, big_ref)
    return big_ref[...]


# speedup vs baseline: 1.0029x; 1.0029x over previous
import jax
import jax.numpy as jnp
from jax import lax
from jax.experimental import pallas as pl
from jax.experimental.pallas import tpu as pltpu
from jax.experimental.pallas import tpu_sc as plsc

B, T, S, D = 32, 8, 2048, 1024
H = D // 2
L = 16
SBLK = 1024
NS = S // SBLK
NUM_SC_CORES = 2


def _rope_sc_body(k_new_hbm, v_new_hbm, cos_hbm, sin_hbm, pos_hbm, big_ref,
                  pos_v, kv, vv, crow, srow, rot, sem):
    c = lax.axis_index("c")
    s = lax.axis_index("s")
    b = s * NUM_SC_CORES + c

    pltpu.sync_copy(pos_hbm, pos_v)
    idx = pos_v.at[b]
    cp_c = pltpu.async_copy(cos_hbm.at[idx], crow, sem)
    cp_s = pltpu.async_copy(sin_hbm.at[idx], srow, sem)
    cp_v = pltpu.async_copy(v_new_hbm.at[b], vv, sem)
    pltpu.sync_copy(k_new_hbm.at[b], kv)
    cp_c.wait()
    cp_s.wait()

    lane = lax.iota(jnp.int32, L)
    lane2 = lane * 2

    for r in range(T):
        rows = jnp.full((L,), r, jnp.int32)

        def chunk(j, carry, r=r, rows=rows):
            i0 = j * L
            c = crow[r, pl.ds(i0, L)]
            s = srow[r, pl.ds(i0, L)]
            ev = 2 * i0 + lane2
            od = ev + 1
            x_e = plsc.load_gather(kv, [rows, ev])
            x_o = plsc.load_gather(kv, [rows, od])
            plsc.store_scatter(rot, [rows, ev], x_e * c - x_o * s)
            plsc.store_scatter(rot, [rows, od], x_e * s + x_o * c)
            return carry

        lax.fori_loop(0, H // L, chunk, 0)

    cp_v.wait()
    cp_rk = pltpu.async_copy(rot, big_ref.at[0, b].at[idx], sem)
    cp_rv = pltpu.async_copy(vv, big_ref.at[1, b].at[idx], sem)
    cp_rk.wait()
    cp_rv.wait()


def _rope_scatter_sc(k_new, v_new, cos, sin, positions, big_ref):
    mesh = plsc.VectorSubcoreMesh(core_axis_name="c", subcore_axis_name="s")
    fn = pl.kernel(
        _rope_sc_body,
        out_type=(),
        mesh=mesh,
        scratch_types=[
            pltpu.VMEM((B, T), jnp.int32),
            pltpu.VMEM((T, D), jnp.float32),
            pltpu.VMEM((T, D), jnp.float32),
            pltpu.VMEM((T, H), jnp.float32),
            pltpu.VMEM((T, H), jnp.float32),
            pltpu.VMEM((T, D), jnp.float32),
            pltpu.SemaphoreType.DMA,
        ],
        compiler_params=pltpu.CompilerParams(needs_layout_passes=False),
    )
    fn(k_new, v_new, cos, sin, positions, big_ref)


def _copy_body(ck_ref, cv_ref, out_ref):
    out_ref[0, 0] = ck_ref[0]
    out_ref[1, 0] = cv_ref[0]


def _copy_tc(cache_k, cache_v):
    return pl.pallas_call(
        _copy_body,
        grid=(B, NS),
        in_specs=[
            pl.BlockSpec((1, SBLK, D), lambda b, si: (b, si, 0)),
            pl.BlockSpec((1, SBLK, D), lambda b, si: (b, si, 0)),
        ],
        out_specs=pl.BlockSpec((2, 1, SBLK, D), lambda b, si: (0, b, si, 0)),
        out_shape=jax.ShapeDtypeStruct((2, B, S, D), jnp.float32),
        compiler_params=pltpu.CompilerParams(
            dimension_semantics=("arbitrary", "arbitrary"),
        ),
    )(cache_k, cache_v)


def kernel(k_new, v_new, cos, sin, cache_k, cache_v, positions):
    stacked = _copy_tc(cache_k, cache_v)
    big_ref = jax.new_ref(stacked)
    _rope_scatter_sc(k_new, v_new, cos, sin, positions, big_ref)
    return big_ref[...]
